# baseline (speedup 1.0000x reference)
import functools

import jax
import jax.numpy as jnp
from jax import lax
from jax.experimental import pallas as pl
from jax.experimental.pallas import tpu as pltpu
from jax.experimental.pallas import tpu_sc as plsc

B, S1, S2, K = 1024, 1, 32, 17
D = 128
N = B * S1 * S2
NNEI = K - 1

NC, NS = 2, 16
NW = NC * NS
PPW = N // NW
P = 8
NCH = PPW // P
SCHUNK = 128
NSCH = PPW // SCHUNK
LANES = 16


def _sc_body(emb_h, idxn_h, idxs_h, self_out, nei_out,
             idxn_v, idxs_v, rows, nbuf, srows, gsem, osem, ssem, sosem):
    cid = lax.axis_index("c")
    sid = lax.axis_index("s")
    w = sid * NC + cid
    base = w * PPW

    pltpu.sync_copy(idxn_h.at[pl.ds(w * NCH, NCH)], idxn_v)
    pltpu.sync_copy(idxs_h.at[pl.ds(w * NSCH, NSCH)], idxs_v)

    def fire_nei(c, slot):
        pltpu.async_copy(emb_h.at[idxn_v.at[c]], rows.at[slot], gsem.at[slot])

    def wait_nei(c, slot):
        pltpu.make_async_copy(emb_h.at[idxn_v.at[c]], rows.at[slot],
                              gsem.at[slot]).wait()

    def fire_self(j):
        pltpu.async_copy(emb_h.at[idxs_v.at[j]], srows, ssem)

    def harvest_self(j):
        pltpu.make_async_copy(emb_h.at[idxs_v.at[j]], srows, ssem).wait()
        dst = self_out.at[pl.ds(base + j * SCHUNK, SCHUNK)]
        pltpu.async_copy(srows, dst, sosem)
        pltpu.make_async_copy(srows, dst, sosem).wait()

    def reduce_chunk(slot, c):
        def pos(p, carry):
            for d in range(D // LANES):
                acc = rows[slot, p * NNEI, pl.ds(d * LANES, LANES)]
                for r in range(1, NNEI):
                    acc = acc + rows[slot, p * NNEI + r, pl.ds(d * LANES, LANES)]
                nbuf[slot, p, pl.ds(d * LANES, LANES)] = acc
            return carry
        lax.fori_loop(0, P, pos, 0)

    for s in range(4):
        fire_nei(s, s)
    fire_self(0)

    def process(i, c, slot):
        wait_nei(c, slot)

        @pl.when(i > 0)
        def _():
            pltpu.make_async_copy(nbuf.at[slot],
                                  nei_out.at[pl.ds(base, P)],
                                  osem.at[slot]).wait()

        reduce_chunk(slot, c)
        pltpu.async_copy(nbuf.at[slot],
                         nei_out.at[pl.ds(base + c * P, P)],
                         osem.at[slot])

        @pl.when(i < NCH // 4 - 1)
        def _():
            fire_nei(c + 4, slot)

    def quad_body(i, carry):
        @pl.when(i % 4 == 2)
        def _():
            j = i // 4
            harvest_self(j)

            @pl.when(j + 1 < NSCH)
            def _():
                fire_self(j + 1)

        for s in range(4):
            process(i, 4 * i + s, s)
        return carry

    lax.fori_loop(0, NCH // 4, quad_body, 0)

    for s in range(4):
        pltpu.make_async_copy(nbuf.at[s], nei_out.at[pl.ds(base, P)],
                              osem.at[s]).wait()


_sc_gather_reduce = functools.partial(
    pl.kernel,
    out_type=(jax.ShapeDtypeStruct((N, D), jnp.float32),
              jax.ShapeDtypeStruct((N, D), jnp.float32)),
    mesh=plsc.VectorSubcoreMesh(core_axis_name="c", subcore_axis_name="s"),
    scratch_types=[
        pltpu.VMEM((NCH, 128), jnp.int32),
        pltpu.VMEM((NSCH, 128), jnp.int32),
        pltpu.VMEM((4, P * NNEI, D), jnp.float32),
        pltpu.VMEM((4, P, D), jnp.float32),
        pltpu.VMEM((SCHUNK, D), jnp.float32),
        pltpu.SemaphoreType.DMA((4,)),
        pltpu.SemaphoreType.DMA((4,)),
        pltpu.SemaphoreType.DMA,
        pltpu.SemaphoreType.DMA,
    ],
)(_sc_body)


def _mm_body(self_ref, nei_ref, w0a, w0s, w1a, w1s, out_ref):
    aggr = nei_ref[...] * (1.0 / NNEI)
    h = jnp.maximum(
        jnp.dot(self_ref[...], w0s[...], preferred_element_type=jnp.float32)
        + jnp.dot(aggr, w0a[...], preferred_element_type=jnp.float32), 0.0)
    out_ref[...] = jnp.maximum(
        jnp.dot(h, w1s[...], preferred_element_type=jnp.float32)
        + jnp.dot(aggr, w1a[...], preferred_element_type=jnp.float32), 0.0)


_MM_R = 4096
_mm = pl.pallas_call(
    _mm_body,
    out_shape=jax.ShapeDtypeStruct((N, D), jnp.float32),
    grid=(N // _MM_R,),
    in_specs=[
        pl.BlockSpec((_MM_R, D), lambda i: (i, 0)),
        pl.BlockSpec((_MM_R, D), lambda i: (i, 0)),
        pl.BlockSpec((D, D), lambda i: (0, 0)),
        pl.BlockSpec((D, D), lambda i: (0, 0)),
        pl.BlockSpec((D, D), lambda i: (0, 0)),
        pl.BlockSpec((D, D), lambda i: (0, 0)),
    ],
    out_specs=pl.BlockSpec((_MM_R, D), lambda i: (i, 0)),
)


def kernel(adj_org, Emb, W0_agg, W0_self, W1_agg, W1_self):
    adj = adj_org.reshape(N, K).astype(jnp.int32)
    idx_self = adj[:, 0].reshape(N // 128, 128)
    idx_nei = adj[:, 1:].reshape(N * NNEI // 128, 128)
    self_rows, nei_sum = _sc_gather_reduce(Emb, idx_nei, idx_self)
    out = _mm(self_rows, nei_sum, W0_agg, W0_self, W1_agg, W1_self)
    return out.reshape(B, S1, S2, D)


# speedup vs baseline: 1.1017x; 1.1017x over previous
import functools

import jax
import jax.numpy as jnp
from jax import lax
from jax.experimental import pallas as pl
from jax.experimental.pallas import tpu as pltpu
from jax.experimental.pallas import tpu_sc as plsc

B, S1, S2, K = 1024, 1, 32, 17
D = 128
N = B * S1 * S2
NNEI = K - 1

NC, NS = 2, 16
NW = NC * NS
PPW = N // NW
P = 8
NCH = PPW // P
SCHUNK = 128
NSCH = PPW // SCHUNK
LANES = 16


def _sc_body(emb_h, idxn_h, idxs_h, self_out, nei_out,
             idxn_v, idxs_v, rows, nbuf, srows, gsem, osem, ssem, sosem):
    cid = lax.axis_index("c")
    sid = lax.axis_index("s")
    w = sid * NC + cid
    base = w * PPW

    pltpu.sync_copy(idxn_h.at[pl.ds(w * NCH, NCH)], idxn_v)
    pltpu.sync_copy(idxs_h.at[pl.ds(w * NSCH, NSCH)], idxs_v)

    def fire_nei(c, slot):
        pltpu.async_copy(emb_h.at[idxn_v.at[c]], rows.at[slot], gsem.at[slot])

    def wait_nei(c, slot):
        pltpu.make_async_copy(emb_h.at[idxn_v.at[c]], rows.at[slot],
                              gsem.at[slot]).wait()

    def fire_self(j):
        pltpu.async_copy(emb_h.at[idxs_v.at[j]], srows, ssem)

    def harvest_self(j):
        pltpu.make_async_copy(emb_h.at[idxs_v.at[j]], srows, ssem).wait()
        dst = self_out.at[pl.ds(base + j * SCHUNK, SCHUNK)]
        pltpu.async_copy(srows, dst, sosem)
        pltpu.make_async_copy(srows, dst, sosem).wait()

    def reduce_chunk(slot, c):
        def pos(p, carry):
            for d in range(D // LANES):
                acc = rows[slot, p * NNEI, pl.ds(d * LANES, LANES)]
                for r in range(1, NNEI):
                    acc = acc + rows[slot, p * NNEI + r, pl.ds(d * LANES, LANES)]
                nbuf[slot, p, pl.ds(d * LANES, LANES)] = acc
            return carry
        lax.fori_loop(0, P, pos, 0)

    fire_nei(0, 0)
    fire_nei(1, 1)
    fire_self(0)

    def process(i, c, slot):
        wait_nei(c, slot)

        @pl.when(i > 0)
        def _():
            pltpu.make_async_copy(nbuf.at[slot],
                                  nei_out.at[pl.ds(base, P)],
                                  osem.at[slot]).wait()

        reduce_chunk(slot, c)
        pltpu.async_copy(nbuf.at[slot],
                         nei_out.at[pl.ds(base + c * P, P)],
                         osem.at[slot])

        @pl.when(i < NCH // 2 - 1)
        def _():
            fire_nei(c + 2, slot)

    def pair_body(i, carry):
        @pl.when(i % 8 == 4)
        def _():
            j = i // 8
            harvest_self(j)

            @pl.when(j + 1 < NSCH)
            def _():
                fire_self(j + 1)

        process(i, 2 * i, 0)
        process(i, 2 * i + 1, 1)
        return carry

    lax.fori_loop(0, NCH // 2, pair_body, 0)

    pltpu.make_async_copy(nbuf.at[0], nei_out.at[pl.ds(base, P)],
                          osem.at[0]).wait()
    pltpu.make_async_copy(nbuf.at[1], nei_out.at[pl.ds(base, P)],
                          osem.at[1]).wait()


_sc_gather_reduce = functools.partial(
    pl.kernel,
    out_type=(jax.ShapeDtypeStruct((N, D), jnp.float32),
              jax.ShapeDtypeStruct((N, D), jnp.float32)),
    mesh=plsc.VectorSubcoreMesh(core_axis_name="c", subcore_axis_name="s"),
    scratch_types=[
        pltpu.VMEM((NCH, 128), jnp.int32),
        pltpu.VMEM((NSCH, 128), jnp.int32),
        pltpu.VMEM((2, P * NNEI, D), jnp.float32),
        pltpu.VMEM((2, P, D), jnp.float32),
        pltpu.VMEM((SCHUNK, D), jnp.float32),
        pltpu.SemaphoreType.DMA((2,)),
        pltpu.SemaphoreType.DMA((2,)),
        pltpu.SemaphoreType.DMA,
        pltpu.SemaphoreType.DMA,
    ],
)(_sc_body)


def _mm_body(self_ref, nei_ref, w0a, w0s, w1a, w1s, out_ref):
    aggr = nei_ref[...] * (1.0 / NNEI)
    h = jnp.maximum(
        jnp.dot(self_ref[...], w0s[...], preferred_element_type=jnp.float32)
        + jnp.dot(aggr, w0a[...], preferred_element_type=jnp.float32), 0.0)
    out_ref[...] = jnp.maximum(
        jnp.dot(h, w1s[...], preferred_element_type=jnp.float32)
        + jnp.dot(aggr, w1a[...], preferred_element_type=jnp.float32), 0.0)


_MM_R = 4096
_mm = pl.pallas_call(
    _mm_body,
    out_shape=jax.ShapeDtypeStruct((N, D), jnp.float32),
    grid=(N // _MM_R,),
    in_specs=[
        pl.BlockSpec((_MM_R, D), lambda i: (i, 0)),
        pl.BlockSpec((_MM_R, D), lambda i: (i, 0)),
        pl.BlockSpec((D, D), lambda i: (0, 0)),
        pl.BlockSpec((D, D), lambda i: (0, 0)),
        pl.BlockSpec((D, D), lambda i: (0, 0)),
        pl.BlockSpec((D, D), lambda i: (0, 0)),
    ],
    out_specs=pl.BlockSpec((_MM_R, D), lambda i: (i, 0)),
)


def kernel(adj_org, Emb, W0_agg, W0_self, W1_agg, W1_self):
    adj = adj_org.reshape(N, K).astype(jnp.int32)
    idx_self = adj[:, 0].reshape(N // 128, 128)
    idx_nei = adj[:, 1:].reshape(N * NNEI // 128, 128)
    self_rows, nei_sum = _sc_gather_reduce(Emb, idx_nei, idx_self)
    out = _mm(self_rows, nei_sum, W0_agg, W0_self, W1_agg, W1_self)
    return out.reshape(B, S1, S2, D)
